# baseline (speedup 1.0000x reference)
import jax
import jax.numpy as jnp
from jax import lax
from jax.experimental import pallas as pl
from jax.experimental.pallas import tpu as pltpu


def _body(x_ref, row_ref, col_ref, io_ref, pair_ref, out_ref, comb_ref):
    g = pl.program_id(0)
    b = pl.program_id(1)
    h, w, d4 = comb_ref.shape[0], comb_ref.shape[1], row_ref.shape[1]

    @pl.when(b == 0)
    def _build():
        row_b = lax.broadcast_in_dim(row_ref[...], (h, w, d4), (0, 2))
        col_b = lax.broadcast_in_dim(col_ref[...], (h, w, d4), (1, 2))
        io_b = lax.broadcast_in_dim(io_ref[pl.ds(g % 2, 1), :], (h, w, d4), (1, 2))
        pair_b = lax.broadcast_in_dim(pair_ref[pl.ds(g // 2, 1), :], (h, w, d4), (1, 2))
        comb_ref[...] = jnp.concatenate([row_b, col_b, io_b, pair_b], axis=-1)

    out_ref[...] = x_ref[...] + comb_ref[...]


def kernel(x, row_table, col_table, io_table, pair_table):
    B, G, H, W, D = x.shape
    return pl.pallas_call(
        _body,
        grid=(G, B),
        in_specs=[
            pl.BlockSpec((None, None, H, W, D), lambda g, b: (b, g, 0, 0, 0)),
            pl.BlockSpec(row_table.shape, lambda g, b: (0, 0)),
            pl.BlockSpec(col_table.shape, lambda g, b: (0, 0)),
            pl.BlockSpec(io_table.shape, lambda g, b: (0, 0)),
            pl.BlockSpec(pair_table.shape, lambda g, b: (0, 0)),
        ],
        out_specs=pl.BlockSpec((None, None, H, W, D), lambda g, b: (b, g, 0, 0, 0)),
        out_shape=jax.ShapeDtypeStruct(x.shape, x.dtype),
        scratch_shapes=[pltpu.VMEM((H, W, D), jnp.float32)],
    )(x, row_table, col_table, io_table, pair_table)


# speedup vs baseline: 1.1796x; 1.1796x over previous
import jax
import jax.numpy as jnp
from jax import lax
from jax.experimental import pallas as pl
from jax.experimental.pallas import tpu as pltpu


def _body(x_ref, row_ref, col_ref, io_ref, pair_ref, out_ref, comb_ref):
    g = pl.program_id(0)
    bb = pl.program_id(1)
    h, w, d4 = comb_ref.shape[0], comb_ref.shape[1], row_ref.shape[1]

    @pl.when(bb == 0)
    def _build():
        row_b = lax.broadcast_in_dim(row_ref[...], (h, w, d4), (0, 2))
        col_b = lax.broadcast_in_dim(col_ref[...], (h, w, d4), (1, 2))
        io_b = lax.broadcast_in_dim(io_ref[pl.ds(g % 2, 1), :], (h, w, d4), (1, 2))
        pair_b = lax.broadcast_in_dim(pair_ref[pl.ds(g // 2, 1), :], (h, w, d4), (1, 2))
        comb_ref[...] = jnp.concatenate([row_b, col_b, io_b, pair_b], axis=-1)

    out_ref[...] = x_ref[...] + comb_ref[None]


_BB = 4


def kernel(x, row_table, col_table, io_table, pair_table):
    B, G, H, W, D = x.shape
    return pl.pallas_call(
        _body,
        grid=(G, B // _BB),
        in_specs=[
            pl.BlockSpec((_BB, None, H, W, D), lambda g, bb: (bb, g, 0, 0, 0)),
            pl.BlockSpec(row_table.shape, lambda g, bb: (0, 0)),
            pl.BlockSpec(col_table.shape, lambda g, bb: (0, 0)),
            pl.BlockSpec(io_table.shape, lambda g, bb: (0, 0)),
            pl.BlockSpec(pair_table.shape, lambda g, bb: (0, 0)),
        ],
        out_specs=pl.BlockSpec((_BB, None, H, W, D), lambda g, bb: (bb, g, 0, 0, 0)),
        out_shape=jax.ShapeDtypeStruct(x.shape, x.dtype),
        scratch_shapes=[pltpu.VMEM((H, W, D), jnp.float32)],
    )(x, row_table, col_table, io_table, pair_table)


# speedup vs baseline: 1.1847x; 1.0044x over previous
import jax
import jax.numpy as jnp
from jax import lax
from jax.experimental import pallas as pl
from jax.experimental.pallas import tpu as pltpu


def _body(x_ref, row_ref, col_ref, io_ref, pair_ref, out_ref, comb_ref):
    g = pl.program_id(0)
    bb = pl.program_id(1)
    h, w, d4 = comb_ref.shape[0], comb_ref.shape[1], row_ref.shape[1]

    @pl.when(bb == 0)
    def _build():
        row_b = lax.broadcast_in_dim(row_ref[...], (h, w, d4), (0, 2))
        col_b = lax.broadcast_in_dim(col_ref[...], (h, w, d4), (1, 2))
        io_b = lax.broadcast_in_dim(io_ref[pl.ds(g % 2, 1), :], (h, w, d4), (1, 2))
        pair_b = lax.broadcast_in_dim(pair_ref[pl.ds(g // 2, 1), :], (h, w, d4), (1, 2))
        comb_ref[...] = jnp.concatenate([row_b, col_b, io_b, pair_b], axis=-1)

    out_ref[...] = x_ref[...] + comb_ref[None]


_BB = 8


def kernel(x, row_table, col_table, io_table, pair_table):
    B, G, H, W, D = x.shape
    return pl.pallas_call(
        _body,
        grid=(G, B // _BB),
        in_specs=[
            pl.BlockSpec((_BB, None, H, W, D), lambda g, bb: (bb, g, 0, 0, 0)),
            pl.BlockSpec(row_table.shape, lambda g, bb: (0, 0)),
            pl.BlockSpec(col_table.shape, lambda g, bb: (0, 0)),
            pl.BlockSpec(io_table.shape, lambda g, bb: (0, 0)),
            pl.BlockSpec(pair_table.shape, lambda g, bb: (0, 0)),
        ],
        out_specs=pl.BlockSpec((_BB, None, H, W, D), lambda g, bb: (bb, g, 0, 0, 0)),
        out_shape=jax.ShapeDtypeStruct(x.shape, x.dtype),
        scratch_shapes=[pltpu.VMEM((H, W, D), jnp.float32)],
    )(x, row_table, col_table, io_table, pair_table)
